# Initial kernel scaffold; baseline (speedup 1.0000x reference)
import jax
import jax.numpy as jnp
from jax.experimental import pallas as pl


def kernel(xyz, params):
    raise NotImplementedError("write your pallas kernel here")


# speedup vs baseline: 1.4625x; 1.4625x over previous
import functools

import jax
import jax.numpy as jnp
from jax.experimental import pallas as pl


def _fps_body(npoint, xyz_ref, cent_ref):
    x0 = xyz_ref[:, 0, :]
    x1 = xyz_ref[:, 1, :]
    x2 = xyz_ref[:, 2, :]
    B, N = x0.shape
    lane = jax.lax.broadcasted_iota(jnp.int32, (B, N), 1)
    col = jax.lax.broadcasted_iota(jnp.int32, (B, npoint), 1)

    cent_ref[...] = jnp.zeros((B, npoint), jnp.int32)

    def body(i, carry):
        dist, far = carry
        cent_ref[...] += far * (col == i).astype(jnp.int32)
        sel = lane == far
        c0 = jnp.sum(jnp.where(sel, x0, 0.0), axis=1, keepdims=True)
        c1 = jnp.sum(jnp.where(sel, x1, 0.0), axis=1, keepdims=True)
        c2 = jnp.sum(jnp.where(sel, x2, 0.0), axis=1, keepdims=True)
        d = (x0 - c0) ** 2 + (x1 - c1) ** 2 + (x2 - c2) ** 2
        dist = jnp.minimum(dist, d)
        far = jnp.argmax(dist, axis=1, keepdims=True).astype(jnp.int32)
        return dist, far

    dist0 = jnp.full((B, N), 1e10, jnp.float32)
    far0 = jnp.zeros((B, 1), jnp.int32)
    jax.lax.fori_loop(0, npoint, body, (dist0, far0))


def _fps(xyz_t, npoint):
    B, _, N = xyz_t.shape
    return pl.pallas_call(
        functools.partial(_fps_body, npoint),
        out_shape=jax.ShapeDtypeStruct((B, npoint), jnp.int32),
    )(xyz_t)


def _make_layer(tile_m, cin, cout, m, *, has_ct=False, k_ct=1, has_act=False,
                mixed_mask=False, pool_k=0, emit_y=True):
    grid = (m // tile_m,)

    def body(*refs):
        it = iter(refs)
        x_ref = next(it)
        ct_ref = next(it) if has_ct else None
        sc_ref = next(it) if has_act else None
        sh_ref = next(it) if has_act else None
        mk_ref = next(it) if mixed_mask else None
        w_ref = next(it)
        b_ref = next(it)
        outs = list(it)
        oi = 0
        x = x_ref[...]
        if has_ct:
            g = tile_m // k_ct
            ct = ct_ref[...].reshape(g, 1, cin)
            ct = jnp.broadcast_to(ct, (g, k_ct, cin)).reshape(tile_m, cin)
            x = x + ct
        a = x
        if has_act:
            a = a * sc_ref[...] + sh_ref[...]
            if mixed_mask:
                a = jnp.where(mk_ref[...] > 0.5, jnp.maximum(a, 0.0), a)
            else:
                a = jnp.maximum(a, 0.0)
        y = jax.lax.dot_general(a, w_ref[...], (((1,), (1,)), ((), ())),
                                preferred_element_type=jnp.float32)
        y = y + b_ref[...]
        if emit_y:
            outs[oi][...] = y
            oi += 1
        if pool_k:
            gp = tile_m // pool_k
            pooled = jnp.max(y.reshape(gp, pool_k, cout), axis=1)
            outs[oi][...] = pooled
            oi += 1
        s1 = jnp.sum(y, axis=0, keepdims=True)
        s2 = jnp.sum(y * y, axis=0, keepdims=True)
        s1_ref, s2_ref = outs[oi], outs[oi + 1]

        @pl.when(pl.program_id(0) == 0)
        def _():
            s1_ref[...] = s1
            s2_ref[...] = s2

        @pl.when(pl.program_id(0) != 0)
        def _():
            s1_ref[...] += s1
            s2_ref[...] += s2

    in_specs = [pl.BlockSpec((tile_m, cin), lambda i: (i, 0))]
    if has_ct:
        in_specs.append(pl.BlockSpec((tile_m // k_ct, cin), lambda i: (i, 0)))
    if has_act:
        in_specs += [pl.BlockSpec((1, cin), lambda i: (0, 0))] * 2
    if mixed_mask:
        in_specs.append(pl.BlockSpec((1, cin), lambda i: (0, 0)))
    in_specs += [pl.BlockSpec((cout, cin), lambda i: (0, 0)),
                 pl.BlockSpec((1, cout), lambda i: (0, 0))]
    out_specs, out_shape = [], []
    if emit_y:
        out_specs.append(pl.BlockSpec((tile_m, cout), lambda i: (i, 0)))
        out_shape.append(jax.ShapeDtypeStruct((m, cout), jnp.float32))
    if pool_k:
        out_specs.append(pl.BlockSpec((tile_m // pool_k, cout), lambda i: (i, 0)))
        out_shape.append(jax.ShapeDtypeStruct((m // pool_k, cout), jnp.float32))
    out_specs += [pl.BlockSpec((1, cout), lambda i: (0, 0))] * 2
    out_shape += [jax.ShapeDtypeStruct((1, cout), jnp.float32)] * 2

    return pl.pallas_call(body, grid=grid, in_specs=in_specs,
                          out_specs=out_specs, out_shape=out_shape)


def _stats_only(tile_m, cin, m, *, has_ct=False, k_ct=1):
    grid = (m // tile_m,)

    def body(*refs):
        if has_ct:
            x_ref, ct_ref, s1_ref, s2_ref = refs
        else:
            x_ref, s1_ref, s2_ref = refs
        x = x_ref[...]
        if has_ct:
            g = tile_m // k_ct
            ct = ct_ref[...].reshape(g, 1, cin)
            ct = jnp.broadcast_to(ct, (g, k_ct, cin)).reshape(tile_m, cin)
            x = x + ct
        s1 = jnp.sum(x, axis=0, keepdims=True)
        s2 = jnp.sum(x * x, axis=0, keepdims=True)

        @pl.when(pl.program_id(0) == 0)
        def _():
            s1_ref[...] = s1
            s2_ref[...] = s2

        @pl.when(pl.program_id(0) != 0)
        def _():
            s1_ref[...] += s1
            s2_ref[...] += s2

    in_specs = [pl.BlockSpec((tile_m, cin), lambda i: (i, 0))]
    if has_ct:
        in_specs.append(pl.BlockSpec((tile_m // k_ct, cin), lambda i: (i, 0)))
    return pl.pallas_call(
        body, grid=grid, in_specs=in_specs,
        out_specs=[pl.BlockSpec((1, cin), lambda i: (0, 0))] * 2,
        out_shape=[jax.ShapeDtypeStruct((1, cin), jnp.float32)] * 2)


def _finalize(m, c):
    def body(x_ref, sc_ref, sh_ref, o_ref):
        o_ref[...] = jnp.maximum(x_ref[...] * sc_ref[...] + sh_ref[...], 0.0)

    return pl.pallas_call(
        body, out_shape=jax.ShapeDtypeStruct((m, c), jnp.float32))


def _affine(s1, s2, n, g, beta):
    mean = s1[0] / n
    var = s2[0] / n - mean * mean
    scale = g / jnp.sqrt(var + 1e-5)
    shift = beta - mean * scale
    return scale, shift


_TILE = 1024


def _tile_for(m):
    return _TILE if m % _TILE == 0 else m


def _ball_sets(pts, new_pts, radii, ks):
    sq = (jnp.sum(new_pts ** 2, -1)[:, :, None]
          + jnp.sum(pts ** 2, -1)[:, None, :]
          - 2.0 * jnp.einsum('bnc,bmc->bnm', new_pts, pts))
    N = pts.shape[1]
    gi0 = jnp.broadcast_to(jnp.arange(N, dtype=jnp.int32), sq.shape)
    out = []
    for r, k in zip(radii, ks):
        gi = jnp.where(sq > r * r, N, gi0)
        gi = jnp.sort(gi, axis=-1)[:, :, :k]
        first = jnp.broadcast_to(gi[:, :, :1], gi.shape)
        out.append(jnp.where(gi == N, first, gi))
    return out


def _gather_rows(table, idx):
    B, S, K = idx.shape
    flat = idx.reshape(B, S * K)
    return jnp.take_along_axis(table, flat[..., None], axis=1)


def _msg_branch(src_x, ct_x, idx, p, k, in_scale, in_shift, in_mask):
    B, N, cin = src_x.shape
    S = idx.shape[1]
    W1, b1 = p['W'][0], p['b'][0]
    c1 = W1.shape[0]
    m_src = B * N
    zb = jnp.zeros((1, c1), jnp.float32)
    if in_scale is None:
        g_tab, _, _ = _make_layer(_tile_for(m_src), cin, c1, m_src)(
            src_x.reshape(m_src, cin), W1, zb)
    else:
        g_tab, _, _ = _make_layer(_tile_for(m_src), cin, c1, m_src,
                                  has_act=True, mixed_mask=True)(
            src_x.reshape(m_src, cin), in_scale[None], in_shift[None],
            in_mask[None], W1, zb)
    m_ct = B * S
    W1x = W1[:, cin - 3:]
    ct_neg, _, _ = _make_layer(_tile_for(m_ct), 3, c1, m_ct)(
        (-ct_x).reshape(m_ct, 3), W1x, b1[None])
    g = _gather_rows(g_tab.reshape(B, N, c1), idx).reshape(B * S * k, c1)
    m = B * S * k
    tile = _tile_for(m)
    s1, s2 = _stats_only(tile, c1, m, has_ct=True, k_ct=k)(g, ct_neg)
    sc1, sh1 = _affine(s1, s2, m, p['g'][0], p['beta'][0])
    W2, b2 = p['W'][1], p['b'][1]
    c2 = W2.shape[0]
    y2, t1, t2 = _make_layer(tile, c1, c2, m, has_ct=True, k_ct=k,
                             has_act=True)(
        g, ct_neg, sc1[None], sh1[None], W2, b2[None])
    sc2, sh2 = _affine(t1, t2, m, p['g'][1], p['beta'][1])
    W3, b3 = p['W'][2], p['b'][2]
    c3 = W3.shape[0]
    pooled, u1, u2 = _make_layer(tile, c2, c3, m, has_act=True,
                                 pool_k=k, emit_y=False)(
        y2, sc2[None], sh2[None], W3, b3[None])
    sc3, sh3 = _affine(u1, u2, m, p['g'][2], p['beta'][2])
    return pooled.reshape(B, S, c3), sc3, sh3


_NPOINT1, _RADII1, _K1 = 512, (0.1, 0.2, 0.4), (16, 32, 128)
_NPOINT2, _RADII2, _K2 = 128, (0.2, 0.4, 0.8), (32, 64, 128)


def kernel(xyz, params):
    B, _, N = xyz.shape
    x = jnp.transpose(xyz, (0, 2, 1))

    S1 = _NPOINT1
    idx1 = _fps(xyz, S1)
    new1 = jnp.take_along_axis(x, idx1[..., None], axis=1)
    balls1 = _ball_sets(x, new1, _RADII1, _K1)
    outs, scs, shs = [], [], []
    for bidx, k, p in zip(balls1, _K1, params['sa1']):
        pooled, sc, sh = _msg_branch(x, new1, bidx, p, k, None, None, None)
        outs.append(pooled)
        scs.append(sc)
        shs.append(sh)
    l1_raw = jnp.concatenate(outs, -1)
    l1_scale = jnp.concatenate(scs)
    l1_shift = jnp.concatenate(shs)

    S2 = _NPOINT2
    idx2 = _fps(jnp.transpose(new1, (0, 2, 1)), S2)
    new2 = jnp.take_along_axis(new1, idx2[..., None], axis=1)
    balls2 = _ball_sets(new1, new2, _RADII2, _K2)
    src2 = jnp.concatenate([l1_raw, new1], -1)
    nf1 = l1_scale.shape[0]
    in_scale2 = jnp.concatenate([l1_scale, jnp.ones((3,), jnp.float32)])
    in_shift2 = jnp.concatenate([l1_shift, jnp.zeros((3,), jnp.float32)])
    in_mask2 = jnp.concatenate([jnp.ones((nf1,), jnp.float32),
                                jnp.zeros((3,), jnp.float32)])
    outs, scs, shs = [], [], []
    for bidx, k, p in zip(balls2, _K2, params['sa2']):
        pooled, sc, sh = _msg_branch(src2, new2, bidx, p, k,
                                     in_scale2, in_shift2, in_mask2)
        outs.append(pooled)
        scs.append(sc)
        shs.append(sh)
    l2_raw = jnp.concatenate(outs, -1)
    l2_scale = jnp.concatenate(scs)
    l2_shift = jnp.concatenate(shs)

    src3 = jnp.concatenate([new2, l2_raw], -1)
    nf2 = l2_scale.shape[0]
    in_scale3 = jnp.concatenate([jnp.ones((3,), jnp.float32), l2_scale])
    in_shift3 = jnp.concatenate([jnp.zeros((3,), jnp.float32), l2_shift])
    in_mask3 = jnp.concatenate([jnp.zeros((3,), jnp.float32),
                                jnp.ones((nf2,), jnp.float32)])
    p3 = params['sa3']
    m3 = B * S2
    h = src3.reshape(m3, nf2 + 3)
    sc, sh, mk = in_scale3, in_shift3, in_mask3
    for li in range(2):
        W, b = p3['W'][li], p3['b'][li]
        co = W.shape[0]
        h, s1, s2 = _make_layer(_tile_for(m3), h.shape[1], co, m3,
                                has_act=True, mixed_mask=True)(
            h, sc[None], sh[None], mk[None], W, b[None])
        sc, sh = _affine(s1, s2, m3, p3['g'][li], p3['beta'][li])
        mk = jnp.ones((co,), jnp.float32)
    W, b = p3['W'][2], p3['b'][2]
    co = W.shape[0]
    pooled3, s1, s2 = _make_layer(_tile_for(m3), h.shape[1], co, m3,
                                  has_act=True, pool_k=S2, emit_y=False)(
        h, sc[None], sh[None], W, b[None])
    sc, sh = _affine(s1, s2, m3, p3['g'][2], p3['beta'][2])

    h = pooled3
    for name in ['fc1', 'fc2', 'fc3']:
        p = params[name]
        co = p['W'].shape[0]
        h, s1, s2 = _make_layer(B, h.shape[1], co, B, has_act=True)(
            h, sc[None], sh[None], p['W'], p['b'][None])
        sc, sh = _affine(s1, s2, B, p['g'], p['beta'])
    return _finalize(B, h.shape[1])(h, sc[None], sh[None])


# speedup vs baseline: 14.6422x; 10.0121x over previous
import functools

import jax
import jax.numpy as jnp
from jax.experimental import pallas as pl
from jax.experimental.pallas import tpu as pltpu
from jax.experimental.pallas import tpu_sc as plsc


_SC_CORES, _SC_SUBCORES = 2, 16
_SC_NW = _SC_CORES * _SC_SUBCORES


def _sc_gather(table, idx):
    M = idx.shape[0]
    D = table.shape[1]
    chunk = min(32768 // D, M // _SC_NW)
    bpw = M // _SC_NW
    nch = bpw // chunk
    assert bpw * _SC_NW == M and nch * chunk == bpw, (M, D)
    mesh = plsc.VectorSubcoreMesh(core_axis_name="c", subcore_axis_name="s")

    @functools.partial(
        pl.kernel, mesh=mesh,
        out_type=jax.ShapeDtypeStruct((M, D), jnp.float32),
        scratch_types=[pltpu.VMEM((bpw,), jnp.int32),
                       pltpu.VMEM((2, chunk, D), jnp.float32),
                       pltpu.SemaphoreType.DMA((2,)),
                       pltpu.SemaphoreType.DMA((2,))])
    def k(table_hbm, idx_hbm, out_hbm, idx_v, rows_v, sem_g, sem_w):
        wid = jax.lax.axis_index("s") * _SC_CORES + jax.lax.axis_index("c")
        base = wid * bpw
        pltpu.sync_copy(idx_hbm.at[pl.ds(base, bpw)], idx_v)

        def mk_g(j, b):
            return pltpu.make_async_copy(
                table_hbm.at[idx_v.at[pl.ds(j * chunk, chunk)]],
                rows_v.at[b], sem_g.at[b])

        def mk_w(j, b):
            return pltpu.make_async_copy(
                rows_v.at[b], out_hbm.at[pl.ds(base + j * chunk, chunk)],
                sem_w.at[b])

        mk_g(0, 0).start()

        def step(j, carry):
            b = jax.lax.rem(j, 2)

            @pl.when(j + 1 < nch)
            def _():
                @pl.when(j >= 1)
                def _():
                    mk_w(j - 1, 1 - b).wait()

                mk_g(j + 1, 1 - b).start()

            mk_g(j, b).wait()
            mk_w(j, b).start()
            return carry

        jax.lax.fori_loop(0, nch, step, 0)
        mk_w(nch - 1, (nch - 1) % 2).wait()
        if nch > 1:
            mk_w(nch - 2, (nch - 2) % 2).wait()

    return k(table, idx)


def _pin_i32(a):
    def body(x_ref, o_ref):
        o_ref[...] = x_ref[...]

    return pl.pallas_call(
        body, out_shape=jax.ShapeDtypeStruct(a.shape, jnp.int32))(a)


def _fps_body(npoint, xyz_ref, cent_ref):
    x0 = xyz_ref[:, 0, :]
    x1 = xyz_ref[:, 1, :]
    x2 = xyz_ref[:, 2, :]
    B, N = x0.shape
    lane = jax.lax.broadcasted_iota(jnp.int32, (B, N), 1)
    col = jax.lax.broadcasted_iota(jnp.int32, (B, npoint), 1)

    cent_ref[...] = jnp.zeros((B, npoint), jnp.int32)

    def body(i, carry):
        dist, far = carry
        cent_ref[...] += far * (col == i).astype(jnp.int32)
        sel = lane == far
        c0 = jnp.sum(jnp.where(sel, x0, 0.0), axis=1, keepdims=True)
        c1 = jnp.sum(jnp.where(sel, x1, 0.0), axis=1, keepdims=True)
        c2 = jnp.sum(jnp.where(sel, x2, 0.0), axis=1, keepdims=True)
        d = (x0 - c0) ** 2 + (x1 - c1) ** 2 + (x2 - c2) ** 2
        dist = jnp.minimum(dist, d)
        far = jnp.argmax(dist, axis=1, keepdims=True).astype(jnp.int32)
        return dist, far

    dist0 = jnp.full((B, N), 1e10, jnp.float32)
    far0 = jnp.zeros((B, 1), jnp.int32)
    jax.lax.fori_loop(0, npoint, body, (dist0, far0))


def _fps(xyz_t, npoint):
    B, _, N = xyz_t.shape
    return pl.pallas_call(
        functools.partial(_fps_body, npoint),
        out_shape=jax.ShapeDtypeStruct((B, npoint), jnp.int32),
    )(xyz_t)


def _make_layer(tile_m, cin, cout, m, *, has_ct=False, k_ct=1, has_act=False,
                mixed_mask=False, pool_k=0, emit_y=True):
    grid = (m // tile_m,)

    def body(*refs):
        it = iter(refs)
        x_ref = next(it)
        ct_ref = next(it) if has_ct else None
        sc_ref = next(it) if has_act else None
        sh_ref = next(it) if has_act else None
        mk_ref = next(it) if mixed_mask else None
        w_ref = next(it)
        b_ref = next(it)
        outs = list(it)
        oi = 0
        x = x_ref[...]
        if has_ct:
            g = tile_m // k_ct
            ct = ct_ref[...].reshape(g, 1, cin)
            ct = jnp.broadcast_to(ct, (g, k_ct, cin)).reshape(tile_m, cin)
            x = x + ct
        a = x
        if has_act:
            a = a * sc_ref[...] + sh_ref[...]
            if mixed_mask:
                a = jnp.where(mk_ref[...] > 0.5, jnp.maximum(a, 0.0), a)
            else:
                a = jnp.maximum(a, 0.0)
        y = jax.lax.dot_general(a, w_ref[...], (((1,), (1,)), ((), ())),
                                preferred_element_type=jnp.float32)
        y = y + b_ref[...]
        if emit_y:
            outs[oi][...] = y
            oi += 1
        if pool_k:
            gp = tile_m // pool_k
            pooled = jnp.max(y.reshape(gp, pool_k, cout), axis=1)
            outs[oi][...] = pooled
            oi += 1
        s1 = jnp.sum(y, axis=0, keepdims=True)
        s2 = jnp.sum(y * y, axis=0, keepdims=True)
        s1_ref, s2_ref = outs[oi], outs[oi + 1]

        @pl.when(pl.program_id(0) == 0)
        def _():
            s1_ref[...] = s1
            s2_ref[...] = s2

        @pl.when(pl.program_id(0) != 0)
        def _():
            s1_ref[...] += s1
            s2_ref[...] += s2

    in_specs = [pl.BlockSpec((tile_m, cin), lambda i: (i, 0))]
    if has_ct:
        in_specs.append(pl.BlockSpec((tile_m // k_ct, cin), lambda i: (i, 0)))
    if has_act:
        in_specs += [pl.BlockSpec((1, cin), lambda i: (0, 0))] * 2
    if mixed_mask:
        in_specs.append(pl.BlockSpec((1, cin), lambda i: (0, 0)))
    in_specs += [pl.BlockSpec((cout, cin), lambda i: (0, 0)),
                 pl.BlockSpec((1, cout), lambda i: (0, 0))]
    out_specs, out_shape = [], []
    if emit_y:
        out_specs.append(pl.BlockSpec((tile_m, cout), lambda i: (i, 0)))
        out_shape.append(jax.ShapeDtypeStruct((m, cout), jnp.float32))
    if pool_k:
        out_specs.append(pl.BlockSpec((tile_m // pool_k, cout), lambda i: (i, 0)))
        out_shape.append(jax.ShapeDtypeStruct((m // pool_k, cout), jnp.float32))
    out_specs += [pl.BlockSpec((1, cout), lambda i: (0, 0))] * 2
    out_shape += [jax.ShapeDtypeStruct((1, cout), jnp.float32)] * 2

    return pl.pallas_call(body, grid=grid, in_specs=in_specs,
                          out_specs=out_specs, out_shape=out_shape)


def _stats_only(tile_m, cin, m, *, has_ct=False, k_ct=1):
    grid = (m // tile_m,)

    def body(*refs):
        if has_ct:
            x_ref, ct_ref, s1_ref, s2_ref = refs
        else:
            x_ref, s1_ref, s2_ref = refs
        x = x_ref[...]
        if has_ct:
            g = tile_m // k_ct
            ct = ct_ref[...].reshape(g, 1, cin)
            ct = jnp.broadcast_to(ct, (g, k_ct, cin)).reshape(tile_m, cin)
            x = x + ct
        s1 = jnp.sum(x, axis=0, keepdims=True)
        s2 = jnp.sum(x * x, axis=0, keepdims=True)

        @pl.when(pl.program_id(0) == 0)
        def _():
            s1_ref[...] = s1
            s2_ref[...] = s2

        @pl.when(pl.program_id(0) != 0)
        def _():
            s1_ref[...] += s1
            s2_ref[...] += s2

    in_specs = [pl.BlockSpec((tile_m, cin), lambda i: (i, 0))]
    if has_ct:
        in_specs.append(pl.BlockSpec((tile_m // k_ct, cin), lambda i: (i, 0)))
    return pl.pallas_call(
        body, grid=grid, in_specs=in_specs,
        out_specs=[pl.BlockSpec((1, cin), lambda i: (0, 0))] * 2,
        out_shape=[jax.ShapeDtypeStruct((1, cin), jnp.float32)] * 2)


def _finalize(m, c):
    def body(x_ref, sc_ref, sh_ref, o_ref):
        o_ref[...] = jnp.maximum(x_ref[...] * sc_ref[...] + sh_ref[...], 0.0)

    return pl.pallas_call(
        body, out_shape=jax.ShapeDtypeStruct((m, c), jnp.float32))


def _affine(s1, s2, n, g, beta):
    mean = s1[0] / n
    var = s2[0] / n - mean * mean
    scale = g / jnp.sqrt(var + 1e-5)
    shift = beta - mean * scale
    return scale, shift


_TILE = 2048


def _tile_for(m):
    return _TILE if m % _TILE == 0 else m


def _ballq_call(B, S, N, radii, ks, tile_s):
    cb = min(128, N)
    nb = N // cb

    def body(new_ref, xyz_ref, *out_refs):
        new = new_ref[0]
        xyz = xyz_ref[0]
        nn = jnp.sum(new * new, axis=1, keepdims=True)
        xn = jnp.sum(xyz * xyz, axis=0, keepdims=True)
        mm = jax.lax.dot_general(new, xyz, (((1,), (0,)), ((), ())),
                                 preferred_element_type=jnp.float32)
        sq = nn + xn - 2.0 * mm
        r128 = jax.lax.broadcasted_iota(jnp.int32, (cb, cb), 0)
        c128 = jax.lax.broadcasted_iota(jnp.int32, (cb, cb), 1)
        lt128 = (r128 <= c128).astype(jnp.float32)
        rnb = jax.lax.broadcasted_iota(jnp.int32, (nb, nb), 0)
        cnb = jax.lax.broadcasted_iota(jnp.int32, (nb, nb), 1)
        ltnb = (rnb < cnb).astype(jnp.float32)
        for r, k, o_ref in zip(radii, ks, out_refs):
            mask = (sq <= r * r).astype(jnp.float32)
            mb = mask.reshape(tile_s * nb, cb)
            inner = jax.lax.dot_general(mb, lt128, (((1,), (0,)), ((), ())),
                                        preferred_element_type=jnp.float32)
            inner = inner.reshape(tile_s, nb, cb)
            bs = inner[:, :, cb - 1]
            offs = jax.lax.dot_general(bs, ltnb, (((1,), (0,)), ((), ())),
                                       preferred_element_type=jnp.float32)
            rank = (inner + offs[:, :, None]).reshape(tile_s, N)
            total = rank[:, N - 1:N]
            tcol = jax.lax.broadcasted_iota(
                jnp.int32, (tile_s, k), 1).astype(jnp.float32)

            def step(t, cnt):
                tf = t.astype(jnp.float32)
                c = jnp.sum((rank <= tf).astype(jnp.float32), axis=1,
                            keepdims=True)
                return cnt + c * (tcol == tf).astype(jnp.float32)

            cnt = jax.lax.fori_loop(0, k, step,
                                    jnp.zeros((tile_s, k), jnp.float32))
            sel = jnp.where(tcol < total, cnt, cnt[:, 0:1])
            o_ref[0] = sel.astype(jnp.int32)

    grid = (B, S // tile_s)
    return pl.pallas_call(
        body, grid=grid,
        in_specs=[pl.BlockSpec((1, tile_s, 3), lambda b, i: (b, i, 0)),
                  pl.BlockSpec((1, 3, N), lambda b, i: (b, 0, 0))],
        out_specs=[pl.BlockSpec((1, tile_s, k), lambda b, i: (b, i, 0))
                   for k in ks],
        out_shape=[jax.ShapeDtypeStruct((B, S, k), jnp.int32) for k in ks])


def _gather_rows(table, idx):
    B, S, K = idx.shape
    flat = idx.reshape(B, S * K)
    N, C = table.shape[1], table.shape[2]
    M = B * S * K
    gidx = (flat + jnp.arange(B, dtype=jnp.int32)[:, None] * N)
    gidx = _pin_i32(gidx.reshape(M // 128, 128)).reshape(-1)
    tflat = table.reshape(B * N, C)
    out = _sc_gather(tflat, gidx)
    pin = jnp.take_along_axis(table, flat[:, :8, None], axis=1)
    keep0 = jnp.sum(pin) * 0.0
    return out.reshape(B, S * K, C), keep0


def _msg_branch(src_x, ct_x, idx, p, k, in_scale, in_shift, in_mask):
    B, N, cin = src_x.shape
    S = idx.shape[1]
    W1, b1 = p['W'][0], p['b'][0]
    c1 = W1.shape[0]
    c1p = 128
    W1p = jnp.pad(W1, ((0, c1p - c1), (0, 0)))
    b1p = jnp.pad(b1, (0, c1p - c1))
    m_src = B * N
    zb = jnp.zeros((1, c1p), jnp.float32)
    if in_scale is None:
        g_tab, _, _ = _make_layer(_tile_for(m_src), cin, c1p, m_src)(
            src_x.reshape(m_src, cin), W1p, zb)
    else:
        g_tab, _, _ = _make_layer(_tile_for(m_src), cin, c1p, m_src,
                                  has_act=True, mixed_mask=True)(
            src_x.reshape(m_src, cin), in_scale[None], in_shift[None],
            in_mask[None], W1p, zb)
    m_ct = B * S
    W1xp = W1p[:, cin - 3:]
    ct_neg, _, _ = _make_layer(_tile_for(m_ct), 3, c1p, m_ct)(
        (-ct_x).reshape(m_ct, 3), W1xp, b1p[None])
    g, keep0 = _gather_rows(g_tab.reshape(B, N, c1p), idx)
    g = g.reshape(B * S * k, c1p)
    m = B * S * k
    tile = _tile_for(m)
    s1, s2 = _stats_only(tile, c1p, m, has_ct=True, k_ct=k)(g, ct_neg)
    sc1, sh1 = _affine(s1[:, :c1], s2[:, :c1], m, p['g'][0], p['beta'][0])
    sc1 = sc1 + keep0
    sc1 = jnp.pad(sc1, (0, c1p - c1))
    sh1 = jnp.pad(sh1, (0, c1p - c1))
    W2, b2 = p['W'][1], p['b'][1]
    c2 = W2.shape[0]
    W2p = jnp.pad(W2, ((0, 0), (0, c1p - c1)))
    y2, t1, t2 = _make_layer(tile, c1p, c2, m, has_ct=True, k_ct=k,
                             has_act=True)(
        g, ct_neg, sc1[None], sh1[None], W2p, b2[None])
    sc2, sh2 = _affine(t1, t2, m, p['g'][1], p['beta'][1])
    W3, b3 = p['W'][2], p['b'][2]
    c3 = W3.shape[0]
    pooled, u1, u2 = _make_layer(tile, c2, c3, m, has_act=True,
                                 pool_k=k, emit_y=False)(
        y2, sc2[None], sh2[None], W3, b3[None])
    sc3, sh3 = _affine(u1, u2, m, p['g'][2], p['beta'][2])
    return pooled.reshape(B, S, c3), sc3, sh3


_NPOINT1, _RADII1, _K1 = 512, (0.1, 0.2, 0.4), (16, 32, 128)
_NPOINT2, _RADII2, _K2 = 128, (0.2, 0.4, 0.8), (32, 64, 128)


def kernel(xyz, params):
    B, _, N = xyz.shape
    x = jnp.transpose(xyz, (0, 2, 1))

    S1 = _NPOINT1
    idx1 = _fps(xyz, S1)
    new1 = jnp.take_along_axis(x, idx1[..., None], axis=1)
    balls1 = _ballq_call(B, S1, N, _RADII1, _K1, min(256, S1))(new1, xyz)
    outs, scs, shs = [], [], []
    for bidx, k, p in zip(balls1, _K1, params['sa1']):
        pooled, sc, sh = _msg_branch(x, new1, bidx, p, k, None, None, None)
        outs.append(pooled)
        scs.append(sc)
        shs.append(sh)
    l1_raw = jnp.concatenate(outs, -1)
    l1_scale = jnp.concatenate(scs)
    l1_shift = jnp.concatenate(shs)

    S2 = _NPOINT2
    new1_t = jnp.transpose(new1, (0, 2, 1))
    idx2 = _fps(new1_t, S2)
    new2 = jnp.take_along_axis(new1, idx2[..., None], axis=1)
    balls2 = _ballq_call(B, S2, S1, _RADII2, _K2, min(128, S2))(new2, new1_t)
    src2 = jnp.concatenate([l1_raw, new1], -1)
    nf1 = l1_scale.shape[0]
    in_scale2 = jnp.concatenate([l1_scale, jnp.ones((3,), jnp.float32)])
    in_shift2 = jnp.concatenate([l1_shift, jnp.zeros((3,), jnp.float32)])
    in_mask2 = jnp.concatenate([jnp.ones((nf1,), jnp.float32),
                                jnp.zeros((3,), jnp.float32)])
    outs, scs, shs = [], [], []
    for bidx, k, p in zip(balls2, _K2, params['sa2']):
        pooled, sc, sh = _msg_branch(src2, new2, bidx, p, k,
                                     in_scale2, in_shift2, in_mask2)
        outs.append(pooled)
        scs.append(sc)
        shs.append(sh)
    l2_raw = jnp.concatenate(outs, -1)
    l2_scale = jnp.concatenate(scs)
    l2_shift = jnp.concatenate(shs)

    src3 = jnp.concatenate([new2, l2_raw], -1)
    nf2 = l2_scale.shape[0]
    in_scale3 = jnp.concatenate([jnp.ones((3,), jnp.float32), l2_scale])
    in_shift3 = jnp.concatenate([jnp.zeros((3,), jnp.float32), l2_shift])
    in_mask3 = jnp.concatenate([jnp.zeros((3,), jnp.float32),
                                jnp.ones((nf2,), jnp.float32)])
    p3 = params['sa3']
    m3 = B * S2
    h = src3.reshape(m3, nf2 + 3)
    sc, sh, mk = in_scale3, in_shift3, in_mask3
    for li in range(2):
        W, b = p3['W'][li], p3['b'][li]
        co = W.shape[0]
        h, s1, s2 = _make_layer(_tile_for(m3), h.shape[1], co, m3,
                                has_act=True, mixed_mask=True)(
            h, sc[None], sh[None], mk[None], W, b[None])
        sc, sh = _affine(s1, s2, m3, p3['g'][li], p3['beta'][li])
        mk = jnp.ones((co,), jnp.float32)
    W, b = p3['W'][2], p3['b'][2]
    co = W.shape[0]
    pooled3, s1, s2 = _make_layer(_tile_for(m3), h.shape[1], co, m3,
                                  has_act=True, pool_k=S2, emit_y=False)(
        h, sc[None], sh[None], W, b[None])
    sc, sh = _affine(s1, s2, m3, p3['g'][2], p3['beta'][2])

    h = pooled3
    for name in ['fc1', 'fc2', 'fc3']:
        p = params[name]
        co = p['W'].shape[0]
        h, s1, s2 = _make_layer(B, h.shape[1], co, B, has_act=True)(
            h, sc[None], sh[None], p['W'], p['b'][None])
        sc, sh = _affine(s1, s2, B, p['g'], p['beta'])
    return _finalize(B, h.shape[1])(h, sc[None], sh[None])
